# baseline (speedup 1.0000x reference)
import functools

import jax
import jax.numpy as jnp
from jax import lax
from jax.experimental import pallas as pl
from jax.experimental.pallas import tpu as pltpu
from jax.experimental.pallas import tpu_sc as plsc

N = 10000
E = 320000
H = 128
NPAD = 10240

NC = 2
NS = 16
NW = NC * NS
EPW = E // NW
CH = 125
NCH = EPW // CH
GCH = 16
NG = NCH // GCH
NPAIR = GCH // 2
CHD = 80
NCHD = EPW // CHD
RPT = NPAD // NS
ZR = 16
NZC = RPT // ZR

R = 1000
GRID = N // R

_MESH = dict(core_axis_name="c", subcore_axis_name="s", num_cores=NC,
             num_subcores=NS)


def _sc_deg_body(dst_hbm, out_hbm, dstv, onesv, bounce, acc):
    cid = lax.axis_index("c")
    sid = lax.axis_index("s")
    wid = sid * NC + cid

    def _z(i, _):
        bounce[pl.ds(i * 16, 16)] = jnp.zeros((16,), jnp.float32)
        return 0
    lax.fori_loop(0, RPT // 16, _z, 0)
    pltpu.sync_copy(bounce, acc.at[pl.ds(sid * RPT, RPT)])

    def _o(i, _):
        onesv[pl.ds(i * 16, 16)] = jnp.ones((16,), jnp.float32)
        return 0
    lax.fori_loop(0, CHD // 16, _o, 0)

    plsc.subcore_barrier()

    pltpu.sync_copy(dst_hbm.at[wid], dstv)

    def _chunk(j, _):
        pltpu.sync_copy(onesv, acc.at[dstv.at[j]], add=True)
        return 0
    lax.fori_loop(0, NCHD, _chunk, 0)

    plsc.subcore_barrier()

    pltpu.sync_copy(acc.at[pl.ds(sid * RPT, RPT)], bounce)
    pltpu.sync_copy(bounce, out_hbm.at[cid, pl.ds(sid * RPT, RPT)])


_sc_deg = pl.kernel(
    _sc_deg_body,
    out_type=jax.ShapeDtypeStruct((NC, NPAD), jnp.float32),
    mesh=plsc.VectorSubcoreMesh(**_MESH),
    scratch_types=[
        pltpu.VMEM((NCHD, CHD), jnp.int32),
        pltpu.VMEM((CHD,), jnp.float32),
        pltpu.VMEM((RPT,), jnp.float32),
        pltpu.VMEM_SHARED((NPAD,), jnp.float32),
    ],
)


def _sc_agg_body(zp_hbm, src_hbm, dst_hbm, zeros_hbm, out_hbm, srcv, dstv,
                 rows_a, rows_b, acc, sem_ga, sem_gb, sem_sa, sem_sb):
    cid = lax.axis_index("c")
    sid = lax.axis_index("s")
    wid = sid * NC + cid

    pltpu.sync_copy(zeros_hbm, acc.at[pl.ds(sid * RPT, RPT)])

    plsc.subcore_barrier()

    def _drain_scatters():
        pltpu.make_async_copy(rows_a, acc.at[dstv.at[0]], sem_sa).wait()
        pltpu.make_async_copy(rows_b, acc.at[dstv.at[0]], sem_sb).wait()

    def _grp(g, _):
        @pl.when(g > 0)
        def _():
            _drain_scatters()

        pltpu.sync_copy(src_hbm.at[wid, g], srcv)
        pltpu.sync_copy(dst_hbm.at[wid, g], dstv)

        def _pair(t, _):
            @pl.when(t > 0)
            def _():
                _drain_scatters()
            a = 2 * t
            b = 2 * t + 1
            ga = pltpu.async_copy(zp_hbm.at[srcv.at[a]], rows_a, sem_ga)
            gb = pltpu.async_copy(zp_hbm.at[srcv.at[b]], rows_b, sem_gb)
            ga.wait()
            pltpu.async_copy(rows_a, acc.at[dstv.at[a]], sem_sa, add=True)
            gb.wait()
            pltpu.async_copy(rows_b, acc.at[dstv.at[b]], sem_sb, add=True)
            return 0
        lax.fori_loop(0, NPAIR, _pair, 0)
        return 0
    lax.fori_loop(0, NG, _grp, 0)

    _drain_scatters()
    plsc.subcore_barrier()

    pltpu.sync_copy(acc.at[pl.ds(sid * RPT, RPT)],
                    out_hbm.at[cid, pl.ds(sid * RPT, RPT)])


_sc_agg = pl.kernel(
    _sc_agg_body,
    out_type=jax.ShapeDtypeStruct((NC, NPAD, H), jnp.float32),
    mesh=plsc.VectorSubcoreMesh(**_MESH),
    scratch_types=[
        pltpu.VMEM((GCH, CH), jnp.int32),
        pltpu.VMEM((GCH, CH), jnp.int32),
        pltpu.VMEM((CH, H), jnp.float32),
        pltpu.VMEM((CH, H), jnp.float32),
        pltpu.VMEM_SHARED((NPAD, H), jnp.float32),
        pltpu.SemaphoreType.DMA,
        pltpu.SemaphoreType.DMA,
        pltpu.SemaphoreType.DMA,
        pltpu.SemaphoreType.DMA,
    ],
)


_DOT = functools.partial(jnp.dot, preferred_element_type=jnp.float32,
                         precision=lax.Precision.HIGHEST)


def _tc_pre_body(x_ref, d0_ref, d1_ref, w_ref, zp_ref, dinv_ref):
    deg = d0_ref[...] + d1_ref[...] + 1.0
    dinv = lax.rsqrt(deg)
    zp_ref[...] = dinv * _DOT(x_ref[...], w_ref[...])
    dinv_ref[...] = dinv


_tc_pre = pl.pallas_call(
    _tc_pre_body,
    grid=(GRID,),
    in_specs=[
        pl.BlockSpec((R, H), lambda i: (i, 0)),
        pl.BlockSpec((R, 1), lambda i: (i, 0)),
        pl.BlockSpec((R, 1), lambda i: (i, 0)),
        pl.BlockSpec((H, H), lambda i: (0, 0)),
    ],
    out_specs=[
        pl.BlockSpec((R, H), lambda i: (i, 0)),
        pl.BlockSpec((R, 1), lambda i: (i, 0)),
    ],
    out_shape=[
        jax.ShapeDtypeStruct((N, H), jnp.float32),
        jax.ShapeDtypeStruct((N, 1), jnp.float32),
    ],
)


def _layer_h(p_ref, zp_ref, dinv_ref, g_ref, be_ref, rm_ref, rv_ref, b_ref):
    accum = p_ref[0] + p_ref[1] + zp_ref[...]
    dinv = dinv_ref[...]
    pre = dinv * accum + b_ref[...]
    a = g_ref[...] * lax.rsqrt(rv_ref[...] + 1e-5)
    return jnp.maximum(pre * a + (be_ref[...] - rm_ref[...] * a), 0.0)


def _tc_mid0_body(p_ref, zp_ref, dinv_ref, g_ref, be_ref, rm_ref, rv_ref,
                  b_ref, wn_ref, jkw_ref, zpo_ref, jk_ref):
    h = _layer_h(p_ref, zp_ref, dinv_ref, g_ref, be_ref, rm_ref, rv_ref, b_ref)
    zpo_ref[...] = dinv_ref[...] * _DOT(h, wn_ref[...])
    jk_ref[...] = _DOT(h, jkw_ref[...])


def _tc_mid1_body(p_ref, zp_ref, dinv_ref, g_ref, be_ref, rm_ref, rv_ref,
                  b_ref, wn_ref, jkw_ref, jkin_ref, zpo_ref, jk_ref):
    h = _layer_h(p_ref, zp_ref, dinv_ref, g_ref, be_ref, rm_ref, rv_ref, b_ref)
    zpo_ref[...] = dinv_ref[...] * _DOT(h, wn_ref[...])
    jk_ref[...] = jkin_ref[...] + _DOT(h, jkw_ref[...])


def _tc_fin_body(p_ref, zp_ref, dinv_ref, g_ref, be_ref, rm_ref, rv_ref,
                 b_ref, jkw_ref, jkin_ref, jkb_ref, clsw_ref, clsb_ref,
                 out_ref):
    h = _layer_h(p_ref, zp_ref, dinv_ref, g_ref, be_ref, rm_ref, rv_ref, b_ref)
    jk = jkin_ref[...] + _DOT(h, jkw_ref[...]) + jkb_ref[...]
    out_ref[...] = _DOT(jk, clsw_ref[...]) + clsb_ref[...]


def _row_specs():
    return [
        pl.BlockSpec((2, R, H), lambda i: (0, i, 0)),
        pl.BlockSpec((R, H), lambda i: (i, 0)),
        pl.BlockSpec((R, 1), lambda i: (i, 0)),
        pl.BlockSpec((1, H), lambda i: (0, 0)),
        pl.BlockSpec((1, H), lambda i: (0, 0)),
        pl.BlockSpec((1, H), lambda i: (0, 0)),
        pl.BlockSpec((1, H), lambda i: (0, 0)),
        pl.BlockSpec((1, H), lambda i: (0, 0)),
    ]


_W_SPEC = pl.BlockSpec((H, H), lambda i: (0, 0))

_tc_mid0 = pl.pallas_call(
    _tc_mid0_body,
    grid=(GRID,),
    in_specs=_row_specs() + [_W_SPEC, _W_SPEC],
    out_specs=[
        pl.BlockSpec((R, H), lambda i: (i, 0)),
        pl.BlockSpec((R, H), lambda i: (i, 0)),
    ],
    out_shape=[
        jax.ShapeDtypeStruct((N, H), jnp.float32),
        jax.ShapeDtypeStruct((N, H), jnp.float32),
    ],
)

_tc_mid1 = pl.pallas_call(
    _tc_mid1_body,
    grid=(GRID,),
    in_specs=_row_specs() + [_W_SPEC, _W_SPEC,
                             pl.BlockSpec((R, H), lambda i: (i, 0))],
    out_specs=[
        pl.BlockSpec((R, H), lambda i: (i, 0)),
        pl.BlockSpec((R, H), lambda i: (i, 0)),
    ],
    out_shape=[
        jax.ShapeDtypeStruct((N, H), jnp.float32),
        jax.ShapeDtypeStruct((N, H), jnp.float32),
    ],
)

C = 2
_tc_fin = pl.pallas_call(
    _tc_fin_body,
    grid=(GRID,),
    in_specs=_row_specs() + [
        _W_SPEC,
        pl.BlockSpec((R, H), lambda i: (i, 0)),
        pl.BlockSpec((1, H), lambda i: (0, 0)),
        pl.BlockSpec((H, C), lambda i: (0, 0)),
        pl.BlockSpec((1, C), lambda i: (0, 0)),
    ],
    out_specs=pl.BlockSpec((R, C), lambda i: (i, 0)),
    out_shape=jax.ShapeDtypeStruct((N, C), jnp.float32),
)


def kernel(x, edge_index, W0, b0, W1, b1, W2, b2,
           g0, be0, rm0, rv0, g1, be1, rm1, rv1, g2, be2, rm2, rv2,
           jkW, jkb, clsW, clsb):
    src4 = edge_index[0].reshape(NW, NG, GCH, CH)
    dst4 = edge_index[1].reshape(NW, NG, GCH, CH)
    zrows = jnp.zeros((RPT, H), jnp.float32)

    degp = _sc_deg(edge_index[1].reshape(NW, NCHD, CHD))
    zp, dinv = _tc_pre(x, degp[0, :N, None], degp[1, :N, None], W0)

    p = _sc_agg(zp, src4, dst4, zrows)
    zp, jk = _tc_mid0(p, zp, dinv, g0.reshape(1, H), be0.reshape(1, H),
                      rm0.reshape(1, H), rv0.reshape(1, H), b0.reshape(1, H),
                      W1, jkW[:H])

    p = _sc_agg(zp, src4, dst4, zrows)
    zp, jk = _tc_mid1(p, zp, dinv, g1.reshape(1, H), be1.reshape(1, H),
                      rm1.reshape(1, H), rv1.reshape(1, H), b1.reshape(1, H),
                      W2, jkW[H:2 * H], jk)

    p = _sc_agg(zp, src4, dst4, zrows)
    logits = _tc_fin(p, zp, dinv, g2.reshape(1, H), be2.reshape(1, H),
                     rm2.reshape(1, H), rv2.reshape(1, H), b2.reshape(1, H),
                     jkW[2 * H:], jk, jkb.reshape(1, H), clsW,
                     clsb.reshape(1, C))
    return logits


# speedup vs baseline: 1.0658x; 1.0658x over previous
import functools

import jax
import jax.numpy as jnp
from jax import lax
from jax.experimental import pallas as pl
from jax.experimental.pallas import tpu as pltpu
from jax.experimental.pallas import tpu_sc as plsc

N = 10000
E = 320000
H = 128
NPAD = 10240

NC = 2
NS = 16
NW = NC * NS
EPW = E // NW
CH = 125
NCH = EPW // CH
GCH = 16
NG = NCH // GCH
NPAIR = GCH // 2
CHD = 80
NCHD = EPW // CHD
RPT = NPAD // NS
ZR = 16
NZC = RPT // ZR

R = 2000
GRID = N // R

_MESH = dict(core_axis_name="c", subcore_axis_name="s", num_cores=NC,
             num_subcores=NS)


def _sc_deg_body(dst_hbm, out_hbm, dstv, onesv, bounce, acc):
    cid = lax.axis_index("c")
    sid = lax.axis_index("s")
    wid = sid * NC + cid

    def _z(i, _):
        bounce[pl.ds(i * 16, 16)] = jnp.zeros((16,), jnp.float32)
        return 0
    lax.fori_loop(0, RPT // 16, _z, 0)
    pltpu.sync_copy(bounce, acc.at[pl.ds(sid * RPT, RPT)])

    def _o(i, _):
        onesv[pl.ds(i * 16, 16)] = jnp.ones((16,), jnp.float32)
        return 0
    lax.fori_loop(0, CHD // 16, _o, 0)

    plsc.subcore_barrier()

    pltpu.sync_copy(dst_hbm.at[wid], dstv)

    def _chunk(j, _):
        pltpu.sync_copy(onesv, acc.at[dstv.at[j]], add=True)
        return 0
    lax.fori_loop(0, NCHD, _chunk, 0)

    plsc.subcore_barrier()

    pltpu.sync_copy(acc.at[pl.ds(sid * RPT, RPT)], bounce)
    pltpu.sync_copy(bounce, out_hbm.at[cid, pl.ds(sid * RPT, RPT)])


_sc_deg = pl.kernel(
    _sc_deg_body,
    out_type=jax.ShapeDtypeStruct((NC, NPAD), jnp.float32),
    mesh=plsc.VectorSubcoreMesh(**_MESH),
    scratch_types=[
        pltpu.VMEM((NCHD, CHD), jnp.int32),
        pltpu.VMEM((CHD,), jnp.float32),
        pltpu.VMEM((RPT,), jnp.float32),
        pltpu.VMEM_SHARED((NPAD,), jnp.float32),
    ],
)


def _sc_agg_body(zp_hbm, src_hbm, dst_hbm, zeros_hbm, out_hbm, srcv, dstv,
                 rows_a, rows_b, acc, sem_ga, sem_gb, sem_sa, sem_sb):
    cid = lax.axis_index("c")
    sid = lax.axis_index("s")
    wid = sid * NC + cid

    pltpu.sync_copy(zeros_hbm, acc.at[pl.ds(sid * RPT, RPT)])

    plsc.subcore_barrier()

    def _drain_scatters():
        pltpu.make_async_copy(rows_a, acc.at[dstv.at[0]], sem_sa).wait()
        pltpu.make_async_copy(rows_b, acc.at[dstv.at[0]], sem_sb).wait()

    def _grp(g, _):
        @pl.when(g > 0)
        def _():
            _drain_scatters()

        pltpu.sync_copy(src_hbm.at[wid, g], srcv)
        pltpu.sync_copy(dst_hbm.at[wid, g], dstv)

        def _pair(t, _):
            @pl.when(t > 0)
            def _():
                _drain_scatters()
            a = 2 * t
            b = 2 * t + 1
            ga = pltpu.async_copy(zp_hbm.at[srcv.at[a]], rows_a, sem_ga)
            gb = pltpu.async_copy(zp_hbm.at[srcv.at[b]], rows_b, sem_gb)
            ga.wait()
            pltpu.async_copy(rows_a, acc.at[dstv.at[a]], sem_sa, add=True)
            gb.wait()
            pltpu.async_copy(rows_b, acc.at[dstv.at[b]], sem_sb, add=True)
            return 0
        lax.fori_loop(0, NPAIR, _pair, 0)
        return 0
    lax.fori_loop(0, NG, _grp, 0)

    _drain_scatters()
    plsc.subcore_barrier()

    pltpu.sync_copy(acc.at[pl.ds(sid * RPT, RPT)],
                    out_hbm.at[cid, pl.ds(sid * RPT, RPT)])


_sc_agg = pl.kernel(
    _sc_agg_body,
    out_type=jax.ShapeDtypeStruct((NC, NPAD, H), jnp.float32),
    mesh=plsc.VectorSubcoreMesh(**_MESH),
    scratch_types=[
        pltpu.VMEM((GCH, CH), jnp.int32),
        pltpu.VMEM((GCH, CH), jnp.int32),
        pltpu.VMEM((CH, H), jnp.float32),
        pltpu.VMEM((CH, H), jnp.float32),
        pltpu.VMEM_SHARED((NPAD, H), jnp.float32),
        pltpu.SemaphoreType.DMA,
        pltpu.SemaphoreType.DMA,
        pltpu.SemaphoreType.DMA,
        pltpu.SemaphoreType.DMA,
    ],
)


_DOT = functools.partial(jnp.dot, preferred_element_type=jnp.float32,
                         precision=lax.Precision.HIGHEST)


C = 2
JH = 3 * H


def _tc_pre_body(x_ref, d0_ref, d1_ref, w_ref, jkw_ref, jkb_ref, clsw_ref,
                 clsb_ref, zp_ref, dinv_ref, m_ref, c2_ref):
    deg = d0_ref[...] + d1_ref[...] + 1.0
    dinv = lax.rsqrt(deg)
    zp_ref[...] = dinv * _DOT(x_ref[...], w_ref[...])
    dinv_ref[...] = dinv
    m_ref[...] = _DOT(jkw_ref[...], clsw_ref[...])
    c2_ref[...] = _DOT(jkb_ref[...], clsw_ref[...]) + clsb_ref[...]


_tc_pre = pl.pallas_call(
    _tc_pre_body,
    grid=(GRID,),
    in_specs=[
        pl.BlockSpec((R, H), lambda i: (i, 0)),
        pl.BlockSpec((R, 1), lambda i: (i, 0)),
        pl.BlockSpec((R, 1), lambda i: (i, 0)),
        pl.BlockSpec((H, H), lambda i: (0, 0)),
        pl.BlockSpec((JH, H), lambda i: (0, 0)),
        pl.BlockSpec((1, H), lambda i: (0, 0)),
        pl.BlockSpec((H, C), lambda i: (0, 0)),
        pl.BlockSpec((1, C), lambda i: (0, 0)),
    ],
    out_specs=[
        pl.BlockSpec((R, H), lambda i: (i, 0)),
        pl.BlockSpec((R, 1), lambda i: (i, 0)),
        pl.BlockSpec((JH, C), lambda i: (0, 0)),
        pl.BlockSpec((1, C), lambda i: (0, 0)),
    ],
    out_shape=[
        jax.ShapeDtypeStruct((N, H), jnp.float32),
        jax.ShapeDtypeStruct((N, 1), jnp.float32),
        jax.ShapeDtypeStruct((JH, C), jnp.float32),
        jax.ShapeDtypeStruct((1, C), jnp.float32),
    ],
)


def _layer_h(p_ref, zp_ref, dinv_ref, g_ref, be_ref, rm_ref, rv_ref, b_ref):
    accum = p_ref[0] + p_ref[1] + zp_ref[...]
    dinv = dinv_ref[...]
    pre = dinv * accum + b_ref[...]
    a = g_ref[...] * lax.rsqrt(rv_ref[...] + 1e-5)
    return jnp.maximum(pre * a + (be_ref[...] - rm_ref[...] * a), 0.0)


def _tc_mid0_body(p_ref, zp_ref, dinv_ref, g_ref, be_ref, rm_ref, rv_ref,
                  b_ref, wn_ref, m_ref, c2_ref, zpo_ref, l_ref):
    h = _layer_h(p_ref, zp_ref, dinv_ref, g_ref, be_ref, rm_ref, rv_ref, b_ref)
    zpo_ref[...] = dinv_ref[...] * _DOT(h, wn_ref[...])
    l_ref[...] = _DOT(h, m_ref[...]) + c2_ref[...]


def _tc_mid1_body(p_ref, zp_ref, dinv_ref, g_ref, be_ref, rm_ref, rv_ref,
                  b_ref, wn_ref, m_ref, lin_ref, zpo_ref, l_ref):
    h = _layer_h(p_ref, zp_ref, dinv_ref, g_ref, be_ref, rm_ref, rv_ref, b_ref)
    zpo_ref[...] = dinv_ref[...] * _DOT(h, wn_ref[...])
    l_ref[...] = lin_ref[...] + _DOT(h, m_ref[...])


def _tc_fin_body(p_ref, zp_ref, dinv_ref, g_ref, be_ref, rm_ref, rv_ref,
                 b_ref, m_ref, lin_ref, out_ref):
    h = _layer_h(p_ref, zp_ref, dinv_ref, g_ref, be_ref, rm_ref, rv_ref, b_ref)
    out_ref[...] = lin_ref[...] + _DOT(h, m_ref[...])


def _row_specs():
    return [
        pl.BlockSpec((2, R, H), lambda i: (0, i, 0)),
        pl.BlockSpec((R, H), lambda i: (i, 0)),
        pl.BlockSpec((R, 1), lambda i: (i, 0)),
        pl.BlockSpec((1, H), lambda i: (0, 0)),
        pl.BlockSpec((1, H), lambda i: (0, 0)),
        pl.BlockSpec((1, H), lambda i: (0, 0)),
        pl.BlockSpec((1, H), lambda i: (0, 0)),
        pl.BlockSpec((1, H), lambda i: (0, 0)),
    ]


_W_SPEC = pl.BlockSpec((H, H), lambda i: (0, 0))
_M_SPEC = pl.BlockSpec((H, C), lambda i: (0, 0))
_C2_SPEC = pl.BlockSpec((1, C), lambda i: (0, 0))
_L_SPEC = pl.BlockSpec((R, C), lambda i: (i, 0))

_tc_mid0 = pl.pallas_call(
    _tc_mid0_body,
    grid=(GRID,),
    in_specs=_row_specs() + [_W_SPEC, _M_SPEC, _C2_SPEC],
    out_specs=[
        pl.BlockSpec((R, H), lambda i: (i, 0)),
        _L_SPEC,
    ],
    out_shape=[
        jax.ShapeDtypeStruct((N, H), jnp.float32),
        jax.ShapeDtypeStruct((N, C), jnp.float32),
    ],
)

_tc_mid1 = pl.pallas_call(
    _tc_mid1_body,
    grid=(GRID,),
    in_specs=_row_specs() + [_W_SPEC, _M_SPEC, _L_SPEC],
    out_specs=[
        pl.BlockSpec((R, H), lambda i: (i, 0)),
        _L_SPEC,
    ],
    out_shape=[
        jax.ShapeDtypeStruct((N, H), jnp.float32),
        jax.ShapeDtypeStruct((N, C), jnp.float32),
    ],
)

_tc_fin = pl.pallas_call(
    _tc_fin_body,
    grid=(GRID,),
    in_specs=_row_specs() + [_M_SPEC, _L_SPEC],
    out_specs=pl.BlockSpec((R, C), lambda i: (i, 0)),
    out_shape=jax.ShapeDtypeStruct((N, C), jnp.float32),
)


def kernel(x, edge_index, W0, b0, W1, b1, W2, b2,
           g0, be0, rm0, rv0, g1, be1, rm1, rv1, g2, be2, rm2, rv2,
           jkW, jkb, clsW, clsb):
    src4 = edge_index[0].reshape(NW, NG, GCH, CH)
    dst4 = edge_index[1].reshape(NW, NG, GCH, CH)
    zrows = jnp.zeros((RPT, H), jnp.float32)

    degp = _sc_deg(edge_index[1].reshape(NW, NCHD, CHD))
    zp, dinv, m, c2 = _tc_pre(x, degp[0, :N, None], degp[1, :N, None], W0,
                              jkW, jkb.reshape(1, H), clsW,
                              clsb.reshape(1, C))

    p = _sc_agg(zp, src4, dst4, zrows)
    zp, l = _tc_mid0(p, zp, dinv, g0.reshape(1, H), be0.reshape(1, H),
                     rm0.reshape(1, H), rv0.reshape(1, H), b0.reshape(1, H),
                     W1, m[:H], c2)

    p = _sc_agg(zp, src4, dst4, zrows)
    zp, l = _tc_mid1(p, zp, dinv, g1.reshape(1, H), be1.reshape(1, H),
                     rm1.reshape(1, H), rv1.reshape(1, H), b1.reshape(1, H),
                     W2, m[H:2 * H], l)

    p = _sc_agg(zp, src4, dst4, zrows)
    logits = _tc_fin(p, zp, dinv, g2.reshape(1, H), be2.reshape(1, H),
                     rm2.reshape(1, H), rv2.reshape(1, H), b2.reshape(1, H),
                     m[2 * H:], l)
    return logits


# speedup vs baseline: 1.2472x; 1.1702x over previous
import functools

import jax
import jax.numpy as jnp
from jax import lax
from jax.experimental import pallas as pl
from jax.experimental.pallas import tpu as pltpu
from jax.experimental.pallas import tpu_sc as plsc

N = 10000
E = 320000
H = 128
NPAD = 10240

NC = 2
NS = 16
NW = NC * NS
EPW = E // NW
CH = 50
NCH = EPW // CH
GCH = 40
NG = NCH // GCH
NBUF = 4
NQUAD = GCH // NBUF
CHD = 80
NCHD = EPW // CHD
RPT = NPAD // NS
ZR = 16
NZC = RPT // ZR

R = 2000
GRID = N // R

_MESH = dict(core_axis_name="c", subcore_axis_name="s", num_cores=NC,
             num_subcores=NS)


def _sc_deg_body(dst_hbm, out_hbm, dstv, onesv, bounce, acc):
    cid = lax.axis_index("c")
    sid = lax.axis_index("s")
    wid = sid * NC + cid

    def _z(i, _):
        bounce[pl.ds(i * 16, 16)] = jnp.zeros((16,), jnp.float32)
        return 0
    lax.fori_loop(0, RPT // 16, _z, 0)
    pltpu.sync_copy(bounce, acc.at[pl.ds(sid * RPT, RPT)])

    def _o(i, _):
        onesv[pl.ds(i * 16, 16)] = jnp.ones((16,), jnp.float32)
        return 0
    lax.fori_loop(0, CHD // 16, _o, 0)

    plsc.subcore_barrier()

    pltpu.sync_copy(dst_hbm.at[wid], dstv)

    def _chunk(j, _):
        pltpu.sync_copy(onesv, acc.at[dstv.at[j]], add=True)
        return 0
    lax.fori_loop(0, NCHD, _chunk, 0)

    plsc.subcore_barrier()

    pltpu.sync_copy(acc.at[pl.ds(sid * RPT, RPT)], bounce)
    pltpu.sync_copy(bounce, out_hbm.at[cid, pl.ds(sid * RPT, RPT)])


_sc_deg = pl.kernel(
    _sc_deg_body,
    out_type=jax.ShapeDtypeStruct((NC, NPAD), jnp.float32),
    mesh=plsc.VectorSubcoreMesh(**_MESH),
    scratch_types=[
        pltpu.VMEM((NCHD, CHD), jnp.int32),
        pltpu.VMEM((CHD,), jnp.float32),
        pltpu.VMEM((RPT,), jnp.float32),
        pltpu.VMEM_SHARED((NPAD,), jnp.float32),
    ],
)


def _sc_agg_body(zp_hbm, src_hbm, dst_hbm, zeros_hbm, out_hbm, srcv, dstv,
                 rows0, rows1, rows2, rows3, acc,
                 sg0, sg1, sg2, sg3, ss0, ss1, ss2, ss3):
    cid = lax.axis_index("c")
    sid = lax.axis_index("s")
    wid = sid * NC + cid
    rows = (rows0, rows1, rows2, rows3)
    sgs = (sg0, sg1, sg2, sg3)
    sss = (ss0, ss1, ss2, ss3)

    pltpu.sync_copy(zeros_hbm, acc.at[pl.ds(sid * RPT, RPT)])

    plsc.subcore_barrier()

    def _drain_scatter(q):
        pltpu.make_async_copy(rows[q], acc.at[dstv.at[0]], sss[q]).wait()

    def _grp(g, _):
        @pl.when(g > 0)
        def _():
            for q in range(NBUF):
                _drain_scatter(q)

        pltpu.sync_copy(src_hbm.at[wid, g], srcv)
        pltpu.sync_copy(dst_hbm.at[wid, g], dstv)

        def _quad(t, _):
            for q in range(NBUF):
                @pl.when(t > 0)
                def _():
                    _drain_scatter(q)
                pltpu.async_copy(zp_hbm.at[srcv.at[NBUF * t + q]], rows[q],
                                 sgs[q])
            for q in range(NBUF):
                pltpu.make_async_copy(zp_hbm.at[srcv.at[0]], rows[q],
                                      sgs[q]).wait()
                pltpu.async_copy(rows[q], acc.at[dstv.at[NBUF * t + q]],
                                 sss[q], add=True)
            return 0
        lax.fori_loop(0, NQUAD, _quad, 0)
        return 0
    lax.fori_loop(0, NG, _grp, 0)

    for q in range(NBUF):
        _drain_scatter(q)
    plsc.subcore_barrier()

    pltpu.sync_copy(acc.at[pl.ds(sid * RPT, RPT)],
                    out_hbm.at[cid, pl.ds(sid * RPT, RPT)])


_sc_agg = pl.kernel(
    _sc_agg_body,
    out_type=jax.ShapeDtypeStruct((NC, NPAD, H), jnp.float32),
    mesh=plsc.VectorSubcoreMesh(**_MESH),
    scratch_types=[
        pltpu.VMEM((GCH, CH), jnp.int32),
        pltpu.VMEM((GCH, CH), jnp.int32),
        pltpu.VMEM((CH, H), jnp.float32),
        pltpu.VMEM((CH, H), jnp.float32),
        pltpu.VMEM((CH, H), jnp.float32),
        pltpu.VMEM((CH, H), jnp.float32),
        pltpu.VMEM_SHARED((NPAD, H), jnp.float32),
        pltpu.SemaphoreType.DMA,
        pltpu.SemaphoreType.DMA,
        pltpu.SemaphoreType.DMA,
        pltpu.SemaphoreType.DMA,
        pltpu.SemaphoreType.DMA,
        pltpu.SemaphoreType.DMA,
        pltpu.SemaphoreType.DMA,
        pltpu.SemaphoreType.DMA,
    ],
)


_DOT = functools.partial(jnp.dot, preferred_element_type=jnp.float32,
                         precision=lax.Precision.HIGHEST)


C = 2
JH = 3 * H


def _tc_pre_body(x_ref, d0_ref, d1_ref, w_ref, jkw_ref, jkb_ref, clsw_ref,
                 clsb_ref, zp_ref, dinv_ref, m_ref, c2_ref):
    deg = d0_ref[...] + d1_ref[...] + 1.0
    dinv = lax.rsqrt(deg)
    zp_ref[...] = dinv * _DOT(x_ref[...], w_ref[...])
    dinv_ref[...] = dinv
    m_ref[...] = _DOT(jkw_ref[...], clsw_ref[...])
    c2_ref[...] = _DOT(jkb_ref[...], clsw_ref[...]) + clsb_ref[...]


_tc_pre = pl.pallas_call(
    _tc_pre_body,
    grid=(GRID,),
    in_specs=[
        pl.BlockSpec((R, H), lambda i: (i, 0)),
        pl.BlockSpec((R, 1), lambda i: (i, 0)),
        pl.BlockSpec((R, 1), lambda i: (i, 0)),
        pl.BlockSpec((H, H), lambda i: (0, 0)),
        pl.BlockSpec((JH, H), lambda i: (0, 0)),
        pl.BlockSpec((1, H), lambda i: (0, 0)),
        pl.BlockSpec((H, C), lambda i: (0, 0)),
        pl.BlockSpec((1, C), lambda i: (0, 0)),
    ],
    out_specs=[
        pl.BlockSpec((R, H), lambda i: (i, 0)),
        pl.BlockSpec((R, 1), lambda i: (i, 0)),
        pl.BlockSpec((JH, C), lambda i: (0, 0)),
        pl.BlockSpec((1, C), lambda i: (0, 0)),
    ],
    out_shape=[
        jax.ShapeDtypeStruct((N, H), jnp.float32),
        jax.ShapeDtypeStruct((N, 1), jnp.float32),
        jax.ShapeDtypeStruct((JH, C), jnp.float32),
        jax.ShapeDtypeStruct((1, C), jnp.float32),
    ],
)


def _layer_h(p_ref, zp_ref, dinv_ref, g_ref, be_ref, rm_ref, rv_ref, b_ref):
    accum = p_ref[0] + p_ref[1] + zp_ref[...]
    dinv = dinv_ref[...]
    pre = dinv * accum + b_ref[...]
    a = g_ref[...] * lax.rsqrt(rv_ref[...] + 1e-5)
    return jnp.maximum(pre * a + (be_ref[...] - rm_ref[...] * a), 0.0)


def _tc_mid0_body(p_ref, zp_ref, dinv_ref, g_ref, be_ref, rm_ref, rv_ref,
                  b_ref, wn_ref, m_ref, c2_ref, zpo_ref, l_ref):
    h = _layer_h(p_ref, zp_ref, dinv_ref, g_ref, be_ref, rm_ref, rv_ref, b_ref)
    zpo_ref[...] = dinv_ref[...] * _DOT(h, wn_ref[...])
    l_ref[...] = _DOT(h, m_ref[...]) + c2_ref[...]


def _tc_mid1_body(p_ref, zp_ref, dinv_ref, g_ref, be_ref, rm_ref, rv_ref,
                  b_ref, wn_ref, m_ref, lin_ref, zpo_ref, l_ref):
    h = _layer_h(p_ref, zp_ref, dinv_ref, g_ref, be_ref, rm_ref, rv_ref, b_ref)
    zpo_ref[...] = dinv_ref[...] * _DOT(h, wn_ref[...])
    l_ref[...] = lin_ref[...] + _DOT(h, m_ref[...])


def _tc_fin_body(p_ref, zp_ref, dinv_ref, g_ref, be_ref, rm_ref, rv_ref,
                 b_ref, m_ref, lin_ref, out_ref):
    h = _layer_h(p_ref, zp_ref, dinv_ref, g_ref, be_ref, rm_ref, rv_ref, b_ref)
    out_ref[...] = lin_ref[...] + _DOT(h, m_ref[...])


def _row_specs():
    return [
        pl.BlockSpec((2, R, H), lambda i: (0, i, 0)),
        pl.BlockSpec((R, H), lambda i: (i, 0)),
        pl.BlockSpec((R, 1), lambda i: (i, 0)),
        pl.BlockSpec((1, H), lambda i: (0, 0)),
        pl.BlockSpec((1, H), lambda i: (0, 0)),
        pl.BlockSpec((1, H), lambda i: (0, 0)),
        pl.BlockSpec((1, H), lambda i: (0, 0)),
        pl.BlockSpec((1, H), lambda i: (0, 0)),
    ]


_W_SPEC = pl.BlockSpec((H, H), lambda i: (0, 0))
_M_SPEC = pl.BlockSpec((H, C), lambda i: (0, 0))
_C2_SPEC = pl.BlockSpec((1, C), lambda i: (0, 0))
_L_SPEC = pl.BlockSpec((R, C), lambda i: (i, 0))

_tc_mid0 = pl.pallas_call(
    _tc_mid0_body,
    grid=(GRID,),
    in_specs=_row_specs() + [_W_SPEC, _M_SPEC, _C2_SPEC],
    out_specs=[
        pl.BlockSpec((R, H), lambda i: (i, 0)),
        _L_SPEC,
    ],
    out_shape=[
        jax.ShapeDtypeStruct((N, H), jnp.float32),
        jax.ShapeDtypeStruct((N, C), jnp.float32),
    ],
)

_tc_mid1 = pl.pallas_call(
    _tc_mid1_body,
    grid=(GRID,),
    in_specs=_row_specs() + [_W_SPEC, _M_SPEC, _L_SPEC],
    out_specs=[
        pl.BlockSpec((R, H), lambda i: (i, 0)),
        _L_SPEC,
    ],
    out_shape=[
        jax.ShapeDtypeStruct((N, H), jnp.float32),
        jax.ShapeDtypeStruct((N, C), jnp.float32),
    ],
)

_tc_fin = pl.pallas_call(
    _tc_fin_body,
    grid=(GRID,),
    in_specs=_row_specs() + [_M_SPEC, _L_SPEC],
    out_specs=pl.BlockSpec((R, C), lambda i: (i, 0)),
    out_shape=jax.ShapeDtypeStruct((N, C), jnp.float32),
)


def kernel(x, edge_index, W0, b0, W1, b1, W2, b2,
           g0, be0, rm0, rv0, g1, be1, rm1, rv1, g2, be2, rm2, rv2,
           jkW, jkb, clsW, clsb):
    src4 = edge_index[0].reshape(NW, NG, GCH, CH)
    dst4 = edge_index[1].reshape(NW, NG, GCH, CH)
    zrows = jnp.zeros((RPT, H), jnp.float32)

    degp = _sc_deg(edge_index[1].reshape(NW, NCHD, CHD))
    zp, dinv, m, c2 = _tc_pre(x, degp[0, :N, None], degp[1, :N, None], W0,
                              jkW, jkb.reshape(1, H), clsW,
                              clsb.reshape(1, C))

    p = _sc_agg(zp, src4, dst4, zrows)
    zp, l = _tc_mid0(p, zp, dinv, g0.reshape(1, H), be0.reshape(1, H),
                     rm0.reshape(1, H), rv0.reshape(1, H), b0.reshape(1, H),
                     W1, m[:H], c2)

    p = _sc_agg(zp, src4, dst4, zrows)
    zp, l = _tc_mid1(p, zp, dinv, g1.reshape(1, H), be1.reshape(1, H),
                     rm1.reshape(1, H), rv1.reshape(1, H), b1.reshape(1, H),
                     W2, m[H:2 * H], l)

    p = _sc_agg(zp, src4, dst4, zrows)
    logits = _tc_fin(p, zp, dinv, g2.reshape(1, H), be2.reshape(1, H),
                     rm2.reshape(1, H), rv2.reshape(1, H), b2.reshape(1, H),
                     m[2 * H:], l)
    return logits
